# Initial kernel scaffold; baseline (speedup 1.0000x reference)
import jax
import jax.numpy as jnp
from jax.experimental import pallas as pl


def kernel(x, edge_index, params):
    raise NotImplementedError("write your pallas kernel here")


# speedup vs baseline: 4.8871x; 4.8871x over previous
import functools

import jax
import jax.numpy as jnp
from jax import lax
from jax.experimental import pallas as pl
from jax.experimental.pallas import tpu as pltpu, tpu_sc as plsc

N = 10000
E = 320000
H = 128
NC5 = 5

NCORES = 2
NSUB = 16
NW = NCORES * NSUB

EC = 128
N_ECHUNK = E // EC
ECHUNK_ITERS = -(-N_ECHUNK // NW)

GC = 80
N_GCHUNK = N // GC
GCHUNK_ITERS = -(-N_GCHUNK // NW)

ROW_BLK = 624
ROW_BLK_LAST = N - ROW_BLK * (NSUB - 1)

def _sc_embed_body(emb_hbm, x_hbm, h_hbm, idx_v, rows_v, sem):
    core = lax.axis_index("c")
    sub = lax.axis_index("s")
    wid = sub * NCORES + core

    def body(it, _):
        cid = it * NW + wid

        @pl.when(cid < N_GCHUNK)
        def _():
            base = cid * GC
            pltpu.sync_copy(x_hbm.at[pl.ds(base, GC)], idx_v)
            pltpu.async_copy(emb_hbm.at[idx_v], rows_v, sem).wait()
            pltpu.sync_copy(rows_v, h_hbm.at[pl.ds(base, GC)])

        return ()

    lax.fori_loop(0, GCHUNK_ITERS, body, (), unroll=False)


def _sc_agg_body(h_hbm, src_hbm, dst_hbm, zero_hbm, out_hbm,
                 acc_sh, sidx_v, didx_v, rows_v, sem):
    core = lax.axis_index("c")
    sub = lax.axis_index("s")
    wid = sub * NCORES + core

    row0 = sub * ROW_BLK

    @pl.when((core == 0) & (sub < NSUB - 1))
    def _():
        pltpu.sync_copy(h_hbm.at[pl.ds(row0, ROW_BLK)],
                        acc_sh.at[pl.ds(row0, ROW_BLK)])

    @pl.when((core == 0) & (sub == NSUB - 1))
    def _():
        pltpu.sync_copy(h_hbm.at[pl.ds(row0, ROW_BLK_LAST)],
                        acc_sh.at[pl.ds(row0, ROW_BLK_LAST)])

    @pl.when((core == 1) & (sub < NSUB - 1))
    def _():
        pltpu.sync_copy(zero_hbm.at[pl.ds(row0, ROW_BLK)],
                        acc_sh.at[pl.ds(row0, ROW_BLK)])

    @pl.when((core == 1) & (sub == NSUB - 1))
    def _():
        pltpu.sync_copy(zero_hbm.at[pl.ds(row0, ROW_BLK_LAST)],
                        acc_sh.at[pl.ds(row0, ROW_BLK_LAST)])

    plsc.subcore_barrier()

    def body(it, _):
        cid = it * NW + wid

        @pl.when(cid < N_ECHUNK)
        def _():
            base = cid * EC
            pltpu.sync_copy(src_hbm.at[pl.ds(base, EC)], sidx_v)
            pltpu.sync_copy(dst_hbm.at[pl.ds(base, EC)], didx_v)
            pltpu.async_copy(h_hbm.at[sidx_v], rows_v, sem).wait()
            pltpu.sync_copy(rows_v, acc_sh.at[didx_v], add=True)

        return ()

    lax.fori_loop(0, ECHUNK_ITERS, body, (), unroll=False)

    plsc.subcore_barrier()

    @pl.when(sub < NSUB - 1)
    def _():
        pltpu.sync_copy(acc_sh.at[pl.ds(row0, ROW_BLK)],
                        out_hbm.at[core, pl.ds(row0, ROW_BLK)])

    @pl.when(sub == NSUB - 1)
    def _():
        pltpu.sync_copy(acc_sh.at[pl.ds(row0, ROW_BLK_LAST)],
                        out_hbm.at[core, pl.ds(row0, ROW_BLK_LAST)])


@functools.lru_cache(maxsize=None)
def _sc_kernels():
    mesh = plsc.VectorSubcoreMesh(
        core_axis_name="c", subcore_axis_name="s",
        num_cores=NCORES, num_subcores=NSUB)
    embed = pl.kernel(
        _sc_embed_body,
        out_type=jax.ShapeDtypeStruct((N, H), jnp.float32),
        mesh=mesh,
        scratch_types=[
            pltpu.VMEM((GC,), jnp.int32),
            pltpu.VMEM((GC, H), jnp.float32),
            pltpu.SemaphoreType.DMA,
        ],
    )
    agg = pl.kernel(
        _sc_agg_body,
        out_type=jax.ShapeDtypeStruct((NCORES, N, H), jnp.float32),
        mesh=mesh,
        scratch_types=[
            pltpu.VMEM_SHARED((N, H), jnp.float32),
            pltpu.VMEM((EC,), jnp.int32),
            pltpu.VMEM((EC,), jnp.int32),
            pltpu.VMEM((EC, H), jnp.float32),
            pltpu.SemaphoreType.DMA,
        ],
    )
    return embed, agg


def _bn(z, g, b):
    mu = jnp.mean(z, axis=0, keepdims=True)
    var = jnp.mean((z - mu) ** 2, axis=0, keepdims=True)
    return g * (z - mu) * lax.rsqrt(var + 1e-5) + b


def _dot(a, w):
    return lax.dot_general(a, w, (((1,), (0,)), ((), ())),
                           preferred_element_type=jnp.float32)


def _tc_gin_body(p0, p1, w1, b1, g1, be1, w2, b2, g2, be2, out):
    t = p0[...] + p1[...]
    z = jnp.maximum(_dot(t, w1[...]) + b1[...], 0.0)
    z = _bn(z, g1[...], be1[...])
    z = _dot(z, w2[...]) + b2[...]
    out[...] = _bn(z, g2[...], be2[...])


def _tc_gin(p0, p1, p):
    return pl.pallas_call(
        _tc_gin_body,
        out_shape=jax.ShapeDtypeStruct((N, H), jnp.float32),
    )(p0, p1,
      p["l1"]["W"], p["l1"]["b"].reshape(1, H),
      p["bn1"]["gamma"].reshape(1, H), p["bn1"]["beta"].reshape(1, H),
      p["l2"]["W"], p["l2"]["b"].reshape(1, H),
      p["bn2"]["gamma"].reshape(1, H), p["bn2"]["beta"].reshape(1, H))


def _softplus(x):
    return jnp.maximum(x, 0.0) + jnp.log1p(jnp.exp(-jnp.abs(x)))


def _tc_heads_body(h_ref,
                   hw1, hb1, hg1, hbe1, hw2, hb2, hg2, hbe2,
                   fww, fwb, fmw, fmb, fkw, fkb,
                   vw1, vb1, vg1, vbe1, vw2, vb2, vg2, vbe2,
                   wl_ref, mn_ref, kp_ref, v_ref):
    h = h_ref[...]
    z = _dot(h, hw1[...]) + hb1[...]
    z = _bn(z, hg1[...], hbe1[...])
    z = jnp.maximum(z, 0.0)
    z = _dot(z, hw2[...]) + hb2[...]
    z = _bn(z, hg2[...], hbe2[...])
    wl_ref[...] = _dot(z, fww[...]) + fwb[...]
    mn_ref[...] = jnp.tanh(_dot(z, fmw[...]) + fmb[...]) * 180.0
    kp_ref[...] = _softplus(_dot(z, fkw[...]) + fkb[...])
    v = _dot(h, vw1[...]) + vb1[...]
    v = _bn(v, vg1[...], vbe1[...])
    v = jnp.maximum(v, 0.0)
    v = _dot(v, vw2[...]) + vb2[...]
    v = _bn(v, vg2[...], vbe2[...])
    v_ref[...] = v * 2.0


def _tc_heads(h, p):
    H2, H4 = H // 2, H // 4
    return pl.pallas_call(
        _tc_heads_body,
        out_shape=[
            jax.ShapeDtypeStruct((N, 2 * NC5), jnp.float32),
            jax.ShapeDtypeStruct((N, 2 * NC5), jnp.float32),
            jax.ShapeDtypeStruct((N, 2 * NC5), jnp.float32),
            jax.ShapeDtypeStruct((N, 2), jnp.float32),
        ],
    )(h,
      p["hvm_l1"]["W"], p["hvm_l1"]["b"].reshape(1, H2),
      p["hvm_bn1"]["gamma"].reshape(1, H2), p["hvm_bn1"]["beta"].reshape(1, H2),
      p["hvm_l2"]["W"], p["hvm_l2"]["b"].reshape(1, H2),
      p["hvm_bn2"]["gamma"].reshape(1, H2), p["hvm_bn2"]["beta"].reshape(1, H2),
      p["fc_w"]["W"], p["fc_w"]["b"].reshape(1, 2 * NC5),
      p["fc_m"]["W"], p["fc_m"]["b"].reshape(1, 2 * NC5),
      p["fc_k"]["W"], p["fc_k"]["b"].reshape(1, 2 * NC5),
      p["val_l1"]["W"], p["val_l1"]["b"].reshape(1, H4),
      p["val_bn1"]["gamma"].reshape(1, H4), p["val_bn1"]["beta"].reshape(1, H4),
      p["val_l2"]["W"], p["val_l2"]["b"].reshape(1, 2),
      p["val_bn2"]["gamma"].reshape(1, 2), p["val_bn2"]["beta"].reshape(1, 2))


def kernel(x, edge_index, params):
    src = edge_index[0]
    dst = edge_index[1]
    zeros = jnp.zeros((N, H), jnp.float32)

    sc_embed, sc_agg = _sc_kernels()
    h = sc_embed(params["emb"], x.astype(jnp.int32))
    for i in range(3):
        partials = sc_agg(h, src, dst, zeros)
        h = _tc_gin(partials[0], partials[1], params["gin"][i])

    wl, mn, kp, v = _tc_heads(h, params)
    bsz = wl.shape[0]
    return (wl.reshape(bsz, 2, NC5), mn.reshape(bsz, 2, NC5),
            kp.reshape(bsz, 2, NC5), v[:, 0], v[:, 1])


# speedup vs baseline: 7.3178x; 1.4974x over previous
import functools

import jax
import jax.numpy as jnp
from jax import lax
from jax.experimental import pallas as pl
from jax.experimental.pallas import tpu as pltpu, tpu_sc as plsc

N = 10000
E = 320000
H = 128
NC5 = 5

NCORES = 2
NSUB = 16
NW = NCORES * NSUB

EC = 128
N_ECHUNK = E // EC
ECHUNK_ITERS = -(-N_ECHUNK // NW)

GC = 80
N_GCHUNK = N // GC
GCHUNK_ITERS = -(-N_GCHUNK // NW)

ROW_BLK = 624
ROW_BLK_LAST = N - ROW_BLK * (NSUB - 1)

def _sc_embed_body(emb_hbm, x_hbm, h_hbm, idx_v, rows_v, sem):
    core = lax.axis_index("c")
    sub = lax.axis_index("s")
    wid = sub * NCORES + core

    def body(it, _):
        cid = it * NW + wid

        @pl.when(cid < N_GCHUNK)
        def _():
            base = cid * GC
            pltpu.sync_copy(x_hbm.at[pl.ds(base, GC)], idx_v)
            pltpu.async_copy(emb_hbm.at[idx_v], rows_v, sem).wait()
            pltpu.sync_copy(rows_v, h_hbm.at[pl.ds(base, GC)])

        return ()

    lax.fori_loop(0, GCHUNK_ITERS, body, (), unroll=False)


def _sc_agg_body(h_hbm, src_hbm, dst_hbm, zero_hbm, out_hbm,
                 acc_sh, sidx_v, didx_v, rows_v, sem,
                 sidx2_v, didx2_v, rows2_v, sem2):
    core = lax.axis_index("c")
    sub = lax.axis_index("s")
    wid = sub * NCORES + core

    row0 = sub * ROW_BLK

    @pl.when((core == 0) & (sub < NSUB - 1))
    def _():
        pltpu.sync_copy(h_hbm.at[pl.ds(row0, ROW_BLK)],
                        acc_sh.at[pl.ds(row0, ROW_BLK)])

    @pl.when((core == 0) & (sub == NSUB - 1))
    def _():
        pltpu.sync_copy(h_hbm.at[pl.ds(row0, ROW_BLK_LAST)],
                        acc_sh.at[pl.ds(row0, ROW_BLK_LAST)])

    @pl.when((core == 1) & (sub < NSUB - 1))
    def _():
        pltpu.sync_copy(zero_hbm.at[pl.ds(row0, ROW_BLK)],
                        acc_sh.at[pl.ds(row0, ROW_BLK)])

    @pl.when((core == 1) & (sub == NSUB - 1))
    def _():
        pltpu.sync_copy(zero_hbm.at[pl.ds(row0, ROW_BLK_LAST)],
                        acc_sh.at[pl.ds(row0, ROW_BLK_LAST)])

    plsc.subcore_barrier()

    sidx = (sidx_v, sidx2_v)
    didx = (didx_v, didx2_v)
    rows = (rows_v, rows2_v)
    gsem = (sem, sem2)

    def _load_idx(it, b):
        base = (it * NW + wid) * EC
        pltpu.sync_copy(src_hbm.at[pl.ds(base, EC)], sidx[b])
        pltpu.sync_copy(dst_hbm.at[pl.ds(base, EC)], didx[b])

    def _start_gather(b):
        pltpu.async_copy(h_hbm.at[sidx[b]], rows[b], gsem[b])

    def _valid(it):
        return (it * NW + wid) < N_ECHUNK

    for b in (0, 1):
        _load_idx(b, b)
        _start_gather(b)

    def body(j, _):
        for b in (0, 1):
            it = 2 * j + b

            @pl.when(_valid(it))
            def _():
                pltpu.make_async_copy(h_hbm.at[sidx[b]], rows[b],
                                      gsem[b]).wait()
                pltpu.sync_copy(rows[b], acc_sh.at[didx[b]], add=True)

            @pl.when(_valid(it + 2))
            def _():
                _load_idx(it + 2, b)
                _start_gather(b)

        return ()

    lax.fori_loop(0, (ECHUNK_ITERS + 1) // 2, body, (), unroll=False)

    plsc.subcore_barrier()

    @pl.when(sub < NSUB - 1)
    def _():
        pltpu.sync_copy(acc_sh.at[pl.ds(row0, ROW_BLK)],
                        out_hbm.at[core, pl.ds(row0, ROW_BLK)])

    @pl.when(sub == NSUB - 1)
    def _():
        pltpu.sync_copy(acc_sh.at[pl.ds(row0, ROW_BLK_LAST)],
                        out_hbm.at[core, pl.ds(row0, ROW_BLK_LAST)])


@functools.lru_cache(maxsize=None)
def _sc_kernels():
    mesh = plsc.VectorSubcoreMesh(
        core_axis_name="c", subcore_axis_name="s",
        num_cores=NCORES, num_subcores=NSUB)
    embed = pl.kernel(
        _sc_embed_body,
        out_type=jax.ShapeDtypeStruct((N, H), jnp.float32),
        mesh=mesh,
        scratch_types=[
            pltpu.VMEM((GC,), jnp.int32),
            pltpu.VMEM((GC, H), jnp.float32),
            pltpu.SemaphoreType.DMA,
        ],
    )
    agg = pl.kernel(
        _sc_agg_body,
        out_type=jax.ShapeDtypeStruct((NCORES, N, H), jnp.float32),
        mesh=mesh,
        scratch_types=[
            pltpu.VMEM_SHARED((N, H), jnp.float32),
            pltpu.VMEM((EC,), jnp.int32),
            pltpu.VMEM((EC,), jnp.int32),
            pltpu.VMEM((EC, H), jnp.float32),
            pltpu.SemaphoreType.DMA,
            pltpu.VMEM((EC,), jnp.int32),
            pltpu.VMEM((EC,), jnp.int32),
            pltpu.VMEM((EC, H), jnp.float32),
            pltpu.SemaphoreType.DMA,
        ],
    )
    return embed, agg


def _bn(z, g, b):
    mu = jnp.mean(z, axis=0, keepdims=True)
    var = jnp.mean((z - mu) ** 2, axis=0, keepdims=True)
    return g * (z - mu) * lax.rsqrt(var + 1e-5) + b


def _dot(a, w):
    return lax.dot_general(a, w, (((1,), (0,)), ((), ())),
                           preferred_element_type=jnp.float32)


def _tc_gin_body(p0, p1, w1, b1, g1, be1, w2, b2, g2, be2, out):
    t = p0[...] + p1[...]
    z = jnp.maximum(_dot(t, w1[...]) + b1[...], 0.0)
    z = _bn(z, g1[...], be1[...])
    z = _dot(z, w2[...]) + b2[...]
    out[...] = _bn(z, g2[...], be2[...])


def _tc_gin(p0, p1, p):
    return pl.pallas_call(
        _tc_gin_body,
        out_shape=jax.ShapeDtypeStruct((N, H), jnp.float32),
    )(p0, p1,
      p["l1"]["W"], p["l1"]["b"].reshape(1, H),
      p["bn1"]["gamma"].reshape(1, H), p["bn1"]["beta"].reshape(1, H),
      p["l2"]["W"], p["l2"]["b"].reshape(1, H),
      p["bn2"]["gamma"].reshape(1, H), p["bn2"]["beta"].reshape(1, H))


def _softplus(x):
    return jnp.maximum(x, 0.0) + jnp.log1p(jnp.exp(-jnp.abs(x)))


def _tc_heads_body(h_ref,
                   hw1, hb1, hg1, hbe1, hw2, hb2, hg2, hbe2,
                   fww, fwb, fmw, fmb, fkw, fkb,
                   vw1, vb1, vg1, vbe1, vw2, vb2, vg2, vbe2,
                   wl_ref, mn_ref, kp_ref, v_ref):
    h = h_ref[...]
    z = _dot(h, hw1[...]) + hb1[...]
    z = _bn(z, hg1[...], hbe1[...])
    z = jnp.maximum(z, 0.0)
    z = _dot(z, hw2[...]) + hb2[...]
    z = _bn(z, hg2[...], hbe2[...])
    wl_ref[...] = _dot(z, fww[...]) + fwb[...]
    mn_ref[...] = jnp.tanh(_dot(z, fmw[...]) + fmb[...]) * 180.0
    kp_ref[...] = _softplus(_dot(z, fkw[...]) + fkb[...])
    v = _dot(h, vw1[...]) + vb1[...]
    v = _bn(v, vg1[...], vbe1[...])
    v = jnp.maximum(v, 0.0)
    v = _dot(v, vw2[...]) + vb2[...]
    v = _bn(v, vg2[...], vbe2[...])
    v_ref[...] = v * 2.0


def _tc_heads(h, p):
    H2, H4 = H // 2, H // 4
    return pl.pallas_call(
        _tc_heads_body,
        out_shape=[
            jax.ShapeDtypeStruct((N, 2 * NC5), jnp.float32),
            jax.ShapeDtypeStruct((N, 2 * NC5), jnp.float32),
            jax.ShapeDtypeStruct((N, 2 * NC5), jnp.float32),
            jax.ShapeDtypeStruct((N, 2), jnp.float32),
        ],
    )(h,
      p["hvm_l1"]["W"], p["hvm_l1"]["b"].reshape(1, H2),
      p["hvm_bn1"]["gamma"].reshape(1, H2), p["hvm_bn1"]["beta"].reshape(1, H2),
      p["hvm_l2"]["W"], p["hvm_l2"]["b"].reshape(1, H2),
      p["hvm_bn2"]["gamma"].reshape(1, H2), p["hvm_bn2"]["beta"].reshape(1, H2),
      p["fc_w"]["W"], p["fc_w"]["b"].reshape(1, 2 * NC5),
      p["fc_m"]["W"], p["fc_m"]["b"].reshape(1, 2 * NC5),
      p["fc_k"]["W"], p["fc_k"]["b"].reshape(1, 2 * NC5),
      p["val_l1"]["W"], p["val_l1"]["b"].reshape(1, H4),
      p["val_bn1"]["gamma"].reshape(1, H4), p["val_bn1"]["beta"].reshape(1, H4),
      p["val_l2"]["W"], p["val_l2"]["b"].reshape(1, 2),
      p["val_bn2"]["gamma"].reshape(1, 2), p["val_bn2"]["beta"].reshape(1, 2))


def kernel(x, edge_index, params):
    src = edge_index[0]
    dst = edge_index[1]
    zeros = jnp.zeros((N, H), jnp.float32)

    sc_embed, sc_agg = _sc_kernels()
    h = sc_embed(params["emb"], x.astype(jnp.int32))
    for i in range(3):
        partials = sc_agg(h, src, dst, zeros)
        h = _tc_gin(partials[0], partials[1], params["gin"][i])

    wl, mn, kp, v = _tc_heads(h, params)
    bsz = wl.shape[0]
    return (wl.reshape(bsz, 2, NC5), mn.reshape(bsz, 2, NC5),
            kp.reshape(bsz, 2, NC5), v[:, 0], v[:, 1])
